# baseline (speedup 1.0000x reference)
import jax
import jax.numpy as jnp
from jax import lax
from jax.experimental import pallas as pl
from jax.experimental.pallas import tpu as pltpu
from jax.experimental.pallas import tpu_sc as plsc

N = 50000
E = 800000
NB = 98
BK = 512
NPAD = NB * BK
NT = 32
EPT = E // NT
CH = 128
SUP = 5
NSUP = EPT // (CH * SUP)
TAIL = EPT - NSUP * CH * SUP
SENTB = 127
E_CAP = E + NB * (NT * 8 + CH)
E_ALL = E_CAP + CH
MLOW = 0xFFFF
MKEEP = 0x01FFFFFF

_MESH = dict(core_axis_name="c", subcore_axis_name="s")
_CP = dict(compiler_params=pltpu.CompilerParams(needs_layout_passes=False))


def _wid():
    return lax.axis_index("s") * 2 + lax.axis_index("c")


def _lanes():
    return lax.broadcasted_iota(jnp.int32, (16,), 0)


def _sget(ref, i):
    return plsc.load_gather(ref, [jnp.full((16,), i, jnp.int32)])[0]


def _srl(x, n):
    return lax.shift_right_logical(x, jnp.full(x.shape, n, jnp.int32))


def _hist_body(src_hbm, dst_hbm, counts_hbm, packed_hbm, sbuf, dbuf, pbuf,
               cnt):
    wid = _wid()
    base = wid * EPT
    z16 = jnp.zeros((16,), jnp.int32)
    for g in range(8):
        cnt[pl.ds(g * 16, 16)] = z16

    def count_group(bvec):
        run, last = plsc.scan_count(bvec)
        plsc.addupdate_scatter(cnt, [bvec], run, mask=last)

    def chunk(i, _):
        off = base + i * CH
        pltpu.sync_copy(src_hbm.at[pl.ds(off, CH)], sbuf)
        pltpu.sync_copy(dst_hbm.at[pl.ds(off, CH)], dbuf)

        def grp(g, _):
            cs = pl.ds(g * 16, 16)
            dv = dbuf[cs]
            count_group(_srl(dv, 9))
            pbuf[cs] = sbuf[cs] | lax.shift_left(dv, 16)
            return 0

        lax.fori_loop(0, CH // 16, grp, 0)
        pltpu.sync_copy(pbuf, packed_hbm.at[pl.ds(off, CH)])
        return 0

    lax.fori_loop(0, EPT // CH, chunk, 0)

    off = base + (EPT // CH) * CH
    pltpu.sync_copy(src_hbm.at[pl.ds(off, TAIL)], sbuf.at[pl.ds(0, TAIL)])
    pltpu.sync_copy(dst_hbm.at[pl.ds(off, TAIL)], dbuf.at[pl.ds(0, TAIL)])
    for g in range(3):
        cs = pl.ds(g * 16, 16)
        dv = dbuf[cs]
        bvec = _srl(dv, 9)
        if (g + 1) * 16 > TAIL:
            bvec = jnp.where(_lanes() < TAIL - g * 16, bvec, SENTB)
        count_group(bvec)
        pbuf[cs] = sbuf[cs] | lax.shift_left(dv, 16)
    pltpu.sync_copy(pbuf.at[pl.ds(0, TAIL)],
                    packed_hbm.at[pl.ds(off, TAIL)])

    pltpu.sync_copy(cnt, counts_hbm.at[wid])


def _sc_hist(src_e, dst_e):
    return pl.kernel(
        _hist_body,
        out_type=(
            jax.ShapeDtypeStruct((NT, 128), jnp.int32),
            jax.ShapeDtypeStruct((E,), jnp.int32),
        ),
        mesh=plsc.VectorSubcoreMesh(**_MESH),
        **_CP,
        scratch_types=[
            pltpu.VMEM((CH,), jnp.int32),
            pltpu.VMEM((CH,), jnp.int32),
            pltpu.VMEM((CH,), jnp.int32),
            pltpu.VMEM((128,), jnp.int32),
        ],
    )(src_e, dst_e)


def _scan_counts(cntall, starts_v, caps_v):
    carry = jnp.int32(0)
    for g in range(8):
        cs = pl.ds(g * 16, 16)

        def acc(t, tot):
            return tot + (cntall[t, cs] + 7) // 8 * 8

        tot8 = lax.fori_loop(0, NT, acc, jnp.zeros((16,), jnp.int32))
        capc = (tot8 + (CH - 1)) // CH
        cum = plsc.cumsum(capc)
        starts_v[cs] = cum - capc + carry
        caps_v[cs] = capc
        carry = carry + cum[15]


def _perm_body(packed_hbm, counts_hbm, packedp_hbm,
               cntall, ebig, localbuf, posbuf, goff8_v, loff8_v, len8_v,
               fillS_v, fillN_v, cursorL, gapS_v, gapN_v, semS):
    wid = _wid()
    pltpu.sync_copy(counts_hbm, cntall)

    carry = jnp.int32(0)
    lcarry = jnp.int32(0)
    z16 = jnp.zeros((16,), jnp.int32)
    for g in range(8):
        cs = pl.ds(g * 16, 16)

        def acc(t, tm):
            tot8, mine8 = tm
            v8 = (cntall[t, cs] + 7) // 8 * 8
            return tot8 + v8, mine8 + jnp.where(t < wid, v8, 0)

        tot8, mine8 = lax.fori_loop(0, NT, acc, (z16, z16))
        mycnt = cntall[wid, cs]
        myr8 = (mycnt + 7) // 8 * 8
        cap = (tot8 + (CH - 1)) // CH * CH
        cum = plsc.cumsum(cap)
        gstart = cum - cap + carry
        goff8_v[cs] = _srl(gstart + mine8, 3)
        gapS_v[cs] = gstart + tot8
        gapN_v[cs] = cap - tot8
        carry = carry + cum[15]

        lcum = plsc.cumsum(myr8)
        loff = lcum - myr8 + lcarry
        loff8_v[cs] = _srl(loff, 3)
        len8_v[cs] = _srl(myr8, 3)
        fillS_v[cs] = loff + mycnt
        fillN_v[cs] = myr8 - mycnt
        cursorL[cs] = loff
        lcarry = lcarry + lcum[15]

    def place_group(bvec, vals):
        run, last = plsc.scan_count(bvec)
        basev = plsc.load_gather(cursorL, [bvec])
        plsc.store_scatter(localbuf, [basev + run - 1], vals)
        plsc.addupdate_scatter(cursorL, [bvec], run, mask=last)

    base = wid * EPT

    def superchunk(sc, _):
        off = base + sc * (CH * SUP)
        pltpu.sync_copy(packed_hbm.at[pl.ds(off, CH * SUP)], ebig)

        def grp(g, _):
            cs = pl.ds(g * 16, 16)
            ev = ebig[cs]
            place_group(_srl(ev, 25), ev & MKEEP)
            return 0

        lax.fori_loop(0, CH * SUP // 16, grp, 0)
        return 0

    lax.fori_loop(0, NSUP, superchunk, 0)

    off = base + NSUP * CH * SUP
    pltpu.sync_copy(packed_hbm.at[pl.ds(off, TAIL)], ebig.at[pl.ds(0, TAIL)])
    for g in range(3):
        ev = ebig[pl.ds(g * 16, 16)]
        bvec = _srl(ev, 25)
        if (g + 1) * 16 > TAIL:
            bvec = jnp.where(_lanes() < TAIL - g * 16, bvec, SENTB)
        place_group(bvec, ev & MKEEP)

    dummy = jnp.full((16,), BK << 16, jnp.int32)
    lanes = _lanes()

    def fillb(b, _):
        fs = _sget(fillS_v, b)
        fn = _sget(fillN_v, b)
        plsc.store_scatter(localbuf, [fs + lanes], dummy, mask=lanes < fn)
        return 0

    lax.fori_loop(0, 128, fillb, 0)

    def issue_or_drain(b, do_wait):
        len8 = _sget(len8_v, b)
        lsrc = _sget(loff8_v, b) * 8
        gdst = _sget(goff8_v, b) * 8
        nfull = _srl(len8, 4)

        def dma(src_sl, dst_sl):
            if do_wait:
                pltpu.make_async_copy(localbuf.at[src_sl],
                                      packedp_hbm.at[dst_sl], semS).wait()
            else:
                pltpu.async_copy(localbuf.at[src_sl],
                                 packedp_hbm.at[dst_sl], semS)

        def full(i, _):
            dma(pl.ds(lsrc + i * CH, CH), pl.ds(gdst + i * CH, CH))
            return 0

        lax.fori_loop(0, nfull, full, 0)
        o = nfull * CH
        rem = len8 & 15
        for k in (3, 2, 1, 0):
            n = 8 << k
            szk = lax.shift_right_logical(rem, k) & 1

            @pl.when(szk > 0)
            def _(o=o, n=n):
                dma(pl.ds(lsrc + o, n), pl.ds(gdst + o, n))

            o = o + szk * n

    def blk(bb, _):
        def ib(j, _):
            issue_or_drain(bb * 16 + j, False)
            return 0

        lax.fori_loop(0, 16, ib, 0)

        def db(j, _):
            issue_or_drain(bb * 16 + j, True)
            return 0

        lax.fori_loop(0, 16, db, 0)
        return 0

    lax.fori_loop(0, 8, blk, 0)

    for k in range(4):
        b = wid + k * NT

        @pl.when(b < NB)
        def _(k=k, b=b):
            gs = _sget(gapS_v, b)
            gn = _sget(gapN_v, b)
            for g in range(8):
                jvec = _lanes() + (g * 16)
                posbuf[pl.ds(g * 16, 16)] = jnp.where(
                    jvec < gn, gs + jvec, E_CAP + jvec
                )
                ebig[pl.ds(g * 16, 16)] = dummy
            pltpu.sync_copy(ebig.at[pl.ds(0, CH)],
                            packedp_hbm.at[posbuf])


def _sc_permute(packed_e, counts):
    return pl.kernel(
        _perm_body,
        out_type=jax.ShapeDtypeStruct((E_ALL,), jnp.int32),
        mesh=plsc.VectorSubcoreMesh(**_MESH),
        **_CP,
        scratch_types=[
            pltpu.VMEM((NT, 128), jnp.int32),
            pltpu.VMEM((CH * SUP,), jnp.int32),
            pltpu.VMEM((EPT + 128 * 8,), jnp.int32),
            pltpu.VMEM((CH,), jnp.int32),
            pltpu.VMEM((128,), jnp.int32),
            pltpu.VMEM((128,), jnp.int32),
            pltpu.VMEM((128,), jnp.int32),
            pltpu.VMEM((128,), jnp.int32),
            pltpu.VMEM((128,), jnp.int32),
            pltpu.VMEM((128,), jnp.int32),
            pltpu.VMEM((128,), jnp.int32),
            pltpu.VMEM((128,), jnp.int32),
            pltpu.SemaphoreType.DMA,
        ],
    )(packed_e, counts)


def _deg_body(packedp_hbm, counts_hbm, deg_hbm, cntall, dbuf, starts_v,
              caps_v, slab):
    wid = _wid()
    pltpu.sync_copy(counts_hbm, cntall)
    _scan_counts(cntall, starts_v, caps_v)
    zf = jnp.zeros((16,), jnp.float32)

    for k in range(4):
        b = wid + k * NT

        @pl.when(b < NB)
        def _(b=b):
            def zs(i, _):
                slab[pl.ds(i * 16, 16)] = zf
                return 0

            lax.fori_loop(0, (BK + 32) // 16, zs, 0)
            st = _sget(starts_v, b) * CH
            nch = _sget(caps_v, b)

            def chunk(i, _):
                pltpu.sync_copy(packedp_hbm.at[pl.ds(st + i * CH, CH)], dbuf)

                def grp(g, _):
                    dvec = _srl(dbuf[pl.ds(g * 16, 16)], 16)
                    run, last = plsc.scan_count(dvec)
                    plsc.addupdate_scatter(slab, [dvec],
                                           run.astype(jnp.float32), mask=last)
                    return 0

                lax.fori_loop(0, CH // 16, grp, 0)
                return 0

            lax.fori_loop(0, nch, chunk, 0)

            def outg(g, _):
                slab[pl.ds(g * 16, 16)] = slab[pl.ds(g * 16, 16)] + 1.0
                return 0

            lax.fori_loop(0, BK // 16, outg, 0)
            pltpu.sync_copy(slab.at[pl.ds(0, BK)],
                            deg_hbm.at[pl.ds(b * BK, BK)])


def _sc_deg(packed_p, counts):
    return pl.kernel(
        _deg_body,
        out_type=jax.ShapeDtypeStruct((NPAD,), jnp.float32),
        mesh=plsc.VectorSubcoreMesh(**_MESH),
        **_CP,
        scratch_types=[
            pltpu.VMEM((NT, 128), jnp.int32),
            pltpu.VMEM((CH,), jnp.int32),
            pltpu.VMEM((128,), jnp.int32),
            pltpu.VMEM((128,), jnp.int32),
            pltpu.VMEM((BK + 32,), jnp.float32),
        ],
    )(packed_p, counts)


def _prop_body(packedp_hbm, counts_hbm, g_hbm, dinv_hbm, p_hbm,
               cntall, ebufA, ebufB, idxA, idxB, dlocA, dlocB, msgA, msgB,
               slab, dinvbuf, starts_v, caps_v, semA, semB, semW):
    wid = _wid()
    pltpu.sync_copy(counts_hbm, cntall)
    _scan_counts(cntall, starts_v, caps_v)
    zrow = jnp.zeros((16,), jnp.float32)

    def issue(st, c, eb, ib, db, mb, sm):
        pltpu.sync_copy(packedp_hbm.at[pl.ds(st + c * CH, CH)], eb)
        for g in range(8):
            cs = pl.ds(g * 16, 16)
            ev = eb[cs]
            ib[cs] = ev & MLOW
            db[cs] = _srl(ev, 16)
        pltpu.async_copy(g_hbm.at[ib], mb, sm)

    def drain(ib, mb, sm):
        pltpu.make_async_copy(g_hbm.at[ib], mb, sm).wait()

    def accum(db, mb):
        def grp(g, _):
            dvec = db[pl.ds(g * 16, 16)]
            for l0 in range(0, 16, 2):
                d0 = dvec[l0]
                d1 = dvec[l0 + 1]
                e0 = g * 16 + l0
                v0 = [mb[e0, pl.ds(j * 16, 16)] for j in range(8)]
                v1 = [mb[e0 + 1, pl.ds(j * 16, 16)] for j in range(8)]
                for j in range(8):
                    plsc.addupdate(slab.at[d0, pl.ds(j * 16, 16)], v0[j])
                for j in range(8):
                    plsc.addupdate(slab.at[d1, pl.ds(j * 16, 16)], v1[j])
            return 0

        lax.fori_loop(0, CH // 16, grp, 0)

    def bucket(k, _):
        b = wid + k * NT

        @pl.when(b < NB)
        def _():
            def zs(r, _):
                for j in range(8):
                    slab[r, pl.ds(j * 16, 16)] = zrow
                return 0

            lax.fori_loop(0, BK + 1, zs, 0)

            st = _sget(starts_v, b) * CH
            nch = _sget(caps_v, b)

            @pl.when(nch > 0)
            def _():
                issue(st, 0, ebufA, idxA, dlocA, msgA, semA)

                def pair(ip, _):
                    c0 = ip * 2

                    @pl.when(c0 + 1 < nch)
                    def _():
                        issue(st, c0 + 1, ebufB, idxB, dlocB, msgB, semB)

                    drain(idxA, msgA, semA)
                    accum(dlocA, msgA)

                    @pl.when(c0 + 2 < nch)
                    def _():
                        issue(st, c0 + 2, ebufA, idxA, dlocA, msgA, semA)

                    @pl.when(c0 + 1 < nch)
                    def _():
                        drain(idxB, msgB, semB)
                        accum(dlocB, msgB)

                    return 0

                lax.fori_loop(0, (nch + 1) // 2, pair, 0)

            pltpu.sync_copy(dinv_hbm.at[pl.ds(b * BK, BK)], dinvbuf)
            for c in range(4):
                buf = msgA if c % 2 == 0 else msgB
                if c >= 2:
                    prows = pl.ds(b * BK + (c - 2) * CH, CH)
                    pltpu.make_async_copy(buf, p_hbm.at[prows], semW).wait()

                def rgrp(g, _, c=c, buf=buf):
                    dvvec = dinvbuf[pl.ds(c * CH + g * 16, 16)]
                    for l in range(16):
                        r = g * 16 + l
                        lr = c * CH + r
                        dv = jnp.full((16,), dvvec[l], jnp.float32)
                        sv = [slab[lr, pl.ds(j * 16, 16)] for j in range(8)]
                        for j in range(8):
                            buf[r, pl.ds(j * 16, 16)] = sv[j] * dv
                    return 0

                lax.fori_loop(0, CH // 16, rgrp, 0)
                rows = pl.ds(b * BK + c * CH, CH)
                pltpu.async_copy(buf, p_hbm.at[rows], semW)
            for c in range(2, 4):
                buf = msgA if c % 2 == 0 else msgB
                rows = pl.ds(b * BK + c * CH, CH)
                pltpu.make_async_copy(buf, p_hbm.at[rows], semW).wait()

        return 0

    lax.fori_loop(0, 4, bucket, 0)


def _sc_prop(packed_p, counts, g, dinv):
    return pl.kernel(
        _prop_body,
        out_type=jax.ShapeDtypeStruct((NPAD, 128), jnp.float32),
        mesh=plsc.VectorSubcoreMesh(**_MESH),
        **_CP,
        scratch_types=[
            pltpu.VMEM((NT, 128), jnp.int32),
            pltpu.VMEM((CH,), jnp.int32),
            pltpu.VMEM((CH,), jnp.int32),
            pltpu.VMEM((CH,), jnp.int32),
            pltpu.VMEM((CH,), jnp.int32),
            pltpu.VMEM((CH,), jnp.int32),
            pltpu.VMEM((CH,), jnp.int32),
            pltpu.VMEM((CH, 128), jnp.float32),
            pltpu.VMEM((CH, 128), jnp.float32),
            pltpu.VMEM((BK + 1, 128), jnp.float32),
            pltpu.VMEM((BK,), jnp.float32),
            pltpu.VMEM((128,), jnp.int32),
            pltpu.VMEM((128,), jnp.int32),
            pltpu.SemaphoreType.DMA,
            pltpu.SemaphoreType.DMA,
            pltpu.SemaphoreType.DMA,
        ],
    )(packed_p, counts, g, dinv)


def _t1_body(deg_ref, x_ref, dinv_ref, g0_ref, q0_ref):
    dv = lax.rsqrt(deg_ref[...])
    g0 = x_ref[...] * dv
    dinv_ref[...] = dv
    g0_ref[...] = g0
    q0_ref[...] = g0 * dv


def _tc_stage1(deg2, x_pad):
    return pl.pallas_call(
        _t1_body,
        grid=(NB,),
        in_specs=[
            pl.BlockSpec((BK, 1), lambda i: (i, 0)),
            pl.BlockSpec((BK, 128), lambda i: (i, 0)),
        ],
        out_specs=[
            pl.BlockSpec((BK, 1), lambda i: (i, 0)),
            pl.BlockSpec((BK, 128), lambda i: (i, 0)),
            pl.BlockSpec((BK, 128), lambda i: (i, 0)),
        ],
        out_shape=[
            jax.ShapeDtypeStruct((NPAD, 1), jnp.float32),
            jax.ShapeDtypeStruct((NPAD, 128), jnp.float32),
            jax.ShapeDtypeStruct((NPAD, 128), jnp.float32),
        ],
    )(deg2, x_pad)


def _t2_body(ps_ref, q_ref, w_ref, b_ref, dinv_ref, g_ref, qo_ref):
    p = ps_ref[...] + q_ref[...]
    h = jnp.dot(p, w_ref[...], preferred_element_type=jnp.float32)
    h = jnp.maximum(h + b_ref[...], 0.0)
    dv = dinv_ref[...]
    g = h * dv
    g_ref[...] = g
    qo_ref[...] = g * dv


def _tc_layer(ps, q, w, bvec, dinv2):
    return pl.pallas_call(
        _t2_body,
        grid=(NB,),
        in_specs=[
            pl.BlockSpec((BK, 128), lambda i: (i, 0)),
            pl.BlockSpec((BK, 128), lambda i: (i, 0)),
            pl.BlockSpec((128, 128), lambda i: (0, 0)),
            pl.BlockSpec((1, 128), lambda i: (0, 0)),
            pl.BlockSpec((BK, 1), lambda i: (i, 0)),
        ],
        out_specs=[
            pl.BlockSpec((BK, 128), lambda i: (i, 0)),
            pl.BlockSpec((BK, 128), lambda i: (i, 0)),
        ],
        out_shape=[
            jax.ShapeDtypeStruct((NPAD, 128), jnp.float32),
            jax.ShapeDtypeStruct((NPAD, 128), jnp.float32),
        ],
    )(ps, q, w, bvec, dinv2)


def _t4_body(ps_ref, q_ref, w2_ref, b2_ref, wc_ref, bc_ref, out_ref):
    p = ps_ref[...] + q_ref[...]
    h = jnp.dot(p, w2_ref[...], preferred_element_type=jnp.float32)
    h = jnp.maximum(h + b2_ref[...], 0.0)
    out_ref[...] = (
        jnp.dot(h, wc_ref[...], preferred_element_type=jnp.float32)
        + bc_ref[...]
    )


def _tc_final(ps2, q2, w2, b2v, wcp, bcp):
    return pl.pallas_call(
        _t4_body,
        grid=(NB,),
        in_specs=[
            pl.BlockSpec((BK, 128), lambda i: (i, 0)),
            pl.BlockSpec((BK, 128), lambda i: (i, 0)),
            pl.BlockSpec((128, 128), lambda i: (0, 0)),
            pl.BlockSpec((1, 128), lambda i: (0, 0)),
            pl.BlockSpec((128, 8), lambda i: (0, 0)),
            pl.BlockSpec((1, 8), lambda i: (0, 0)),
        ],
        out_specs=pl.BlockSpec((BK, 8), lambda i: (i, 0)),
        out_shape=jax.ShapeDtypeStruct((NPAD, 8), jnp.float32),
    )(ps2, q2, w2, b2v, wcp, bcp)


def kernel(x, edge_index, W1, b1, W2, b2, Wc, bc):
    src_e = edge_index[0]
    dst_e = edge_index[1]
    x_pad = jnp.pad(x, ((0, NPAD - N), (0, 128 - x.shape[1])))
    W1p = jnp.pad(W1, ((0, 128 - W1.shape[0]), (0, 0)))
    Wcp = jnp.pad(Wc, ((0, 0), (0, 8 - Wc.shape[1])))
    b1r = b1.reshape(1, 128)
    b2r = b2.reshape(1, 128)
    bcp = jnp.pad(bc, (0, 8 - bc.shape[0])).reshape(1, 8)

    counts, packed_e = _sc_hist(src_e, dst_e)
    packed_p = _sc_permute(packed_e, counts)
    deg = _sc_deg(packed_p, counts)
    dinv2, g0, q0 = _tc_stage1(deg.reshape(NPAD, 1), x_pad)
    dinv = dinv2.reshape(NPAD)

    ps0 = _sc_prop(packed_p, counts, g0, dinv)
    g1, q1 = _tc_layer(ps0, q0, W1p, b1r, dinv2)
    ps1 = _sc_prop(packed_p, counts, g1, dinv)
    g2, q2 = _tc_layer(ps1, q1, W2, b2r, dinv2)
    ps2 = _sc_prop(packed_p, counts, g2, dinv)
    out = _tc_final(ps2, q2, W2, b2r, Wcp, bcp)
    return out[:N, :2]


# speedup vs baseline: 1.4077x; 1.4077x over previous
import jax
import jax.numpy as jnp
from jax import lax
from jax.experimental import pallas as pl
from jax.experimental.pallas import tpu as pltpu
from jax.experimental.pallas import tpu_sc as plsc

N = 50000
E = 800000
NB = 98
BK = 512
NPAD = NB * BK
NT = 32
EPT = E // NT
CH = 128
SUP = 5
NSUP = EPT // (CH * SUP)
TAIL = EPT - NSUP * CH * SUP
SENTB = 127
E_CAP = E + NB * (NT * 8 + CH)
E_ALL = E_CAP + CH
MLOW = 0xFFFF
MKEEP = 0x01FFFFFF

_MESH = dict(core_axis_name="c", subcore_axis_name="s")
_CP = dict(compiler_params=pltpu.CompilerParams(needs_layout_passes=False))


def _wid():
    return lax.axis_index("s") * 2 + lax.axis_index("c")


def _lanes():
    return lax.broadcasted_iota(jnp.int32, (16,), 0)


def _sget(ref, i):
    return plsc.load_gather(ref, [jnp.full((16,), i, jnp.int32)])[0]


def _srl(x, n):
    return lax.shift_right_logical(x, jnp.full(x.shape, n, jnp.int32))


def _hist_body(src_hbm, dst_hbm, counts_hbm, packed_hbm, sbuf, dbuf, pbuf,
               cnt):
    wid = _wid()
    base = wid * EPT
    z16 = jnp.zeros((16,), jnp.int32)
    for g in range(8):
        cnt[pl.ds(g * 16, 16)] = z16

    def count_group(bvec):
        run, last = plsc.scan_count(bvec)
        plsc.addupdate_scatter(cnt, [bvec], run, mask=last)

    def chunk(i, _):
        off = base + i * CH
        pltpu.sync_copy(src_hbm.at[pl.ds(off, CH)], sbuf)
        pltpu.sync_copy(dst_hbm.at[pl.ds(off, CH)], dbuf)

        def grp(g, _):
            cs = pl.ds(g * 16, 16)
            dv = dbuf[cs]
            count_group(_srl(dv, 9))
            pbuf[cs] = sbuf[cs] | lax.shift_left(dv, 16)
            return 0

        lax.fori_loop(0, CH // 16, grp, 0)
        pltpu.sync_copy(pbuf, packed_hbm.at[pl.ds(off, CH)])
        return 0

    lax.fori_loop(0, EPT // CH, chunk, 0)

    off = base + (EPT // CH) * CH
    pltpu.sync_copy(src_hbm.at[pl.ds(off, TAIL)], sbuf.at[pl.ds(0, TAIL)])
    pltpu.sync_copy(dst_hbm.at[pl.ds(off, TAIL)], dbuf.at[pl.ds(0, TAIL)])
    for g in range(3):
        cs = pl.ds(g * 16, 16)
        dv = dbuf[cs]
        bvec = _srl(dv, 9)
        if (g + 1) * 16 > TAIL:
            bvec = jnp.where(_lanes() < TAIL - g * 16, bvec, SENTB)
        count_group(bvec)
        pbuf[cs] = sbuf[cs] | lax.shift_left(dv, 16)
    pltpu.sync_copy(pbuf.at[pl.ds(0, TAIL)],
                    packed_hbm.at[pl.ds(off, TAIL)])

    pltpu.sync_copy(cnt, counts_hbm.at[wid])


def _sc_hist(src_e, dst_e):
    return pl.kernel(
        _hist_body,
        out_type=(
            jax.ShapeDtypeStruct((NT, 128), jnp.int32),
            jax.ShapeDtypeStruct((E,), jnp.int32),
        ),
        mesh=plsc.VectorSubcoreMesh(**_MESH),
        **_CP,
        scratch_types=[
            pltpu.VMEM((CH,), jnp.int32),
            pltpu.VMEM((CH,), jnp.int32),
            pltpu.VMEM((CH,), jnp.int32),
            pltpu.VMEM((128,), jnp.int32),
        ],
    )(src_e, dst_e)


def _scan_counts(cntall, starts_v, caps_v):
    carry = jnp.int32(0)
    for g in range(8):
        cs = pl.ds(g * 16, 16)

        def acc(t, tot):
            return tot + (cntall[t, cs] + 7) // 8 * 8

        tot8 = lax.fori_loop(0, NT, acc, jnp.zeros((16,), jnp.int32))
        capc = (tot8 + (CH - 1)) // CH
        cum = plsc.cumsum(capc)
        starts_v[cs] = cum - capc + carry
        caps_v[cs] = capc
        carry = carry + cum[15]


def _perm_body(packed_hbm, counts_hbm, packedp_hbm,
               cntall, ebig, localbuf, posbuf, goff8_v, loff8_v, len8_v,
               fillS_v, fillN_v, cursorL, gapS_v, gapN_v, semS):
    wid = _wid()
    pltpu.sync_copy(counts_hbm, cntall)

    carry = jnp.int32(0)
    lcarry = jnp.int32(0)
    z16 = jnp.zeros((16,), jnp.int32)
    for g in range(8):
        cs = pl.ds(g * 16, 16)

        def acc(t, tm):
            tot8, mine8 = tm
            v8 = (cntall[t, cs] + 7) // 8 * 8
            return tot8 + v8, mine8 + jnp.where(t < wid, v8, 0)

        tot8, mine8 = lax.fori_loop(0, NT, acc, (z16, z16))
        mycnt = cntall[wid, cs]
        myr8 = (mycnt + 7) // 8 * 8
        cap = (tot8 + (CH - 1)) // CH * CH
        cum = plsc.cumsum(cap)
        gstart = cum - cap + carry
        goff8_v[cs] = _srl(gstart + mine8, 3)
        gapS_v[cs] = gstart + tot8
        gapN_v[cs] = cap - tot8
        carry = carry + cum[15]

        lcum = plsc.cumsum(myr8)
        loff = lcum - myr8 + lcarry
        loff8_v[cs] = _srl(loff, 3)
        len8_v[cs] = _srl(myr8, 3)
        fillS_v[cs] = loff + mycnt
        fillN_v[cs] = myr8 - mycnt
        cursorL[cs] = loff
        lcarry = lcarry + lcum[15]

    def place_group(bvec, vals):
        run, last = plsc.scan_count(bvec)
        basev = plsc.load_gather(cursorL, [bvec])
        plsc.store_scatter(localbuf, [basev + run - 1], vals)
        plsc.addupdate_scatter(cursorL, [bvec], run, mask=last)

    base = wid * EPT

    def superchunk(sc, _):
        off = base + sc * (CH * SUP)
        pltpu.sync_copy(packed_hbm.at[pl.ds(off, CH * SUP)], ebig)

        def grp(g, _):
            cs = pl.ds(g * 16, 16)
            ev = ebig[cs]
            place_group(_srl(ev, 25), ev & MKEEP)
            return 0

        lax.fori_loop(0, CH * SUP // 16, grp, 0)
        return 0

    lax.fori_loop(0, NSUP, superchunk, 0)

    off = base + NSUP * CH * SUP
    pltpu.sync_copy(packed_hbm.at[pl.ds(off, TAIL)], ebig.at[pl.ds(0, TAIL)])
    for g in range(3):
        ev = ebig[pl.ds(g * 16, 16)]
        bvec = _srl(ev, 25)
        if (g + 1) * 16 > TAIL:
            bvec = jnp.where(_lanes() < TAIL - g * 16, bvec, SENTB)
        place_group(bvec, ev & MKEEP)

    lanes = _lanes()

    def fillb(b, _):
        fs = _sget(fillS_v, b)
        fn = _sget(fillN_v, b)
        spread = ((fs + lanes + wid * 953) & 0x7FFF) | (BK << 16)
        plsc.store_scatter(localbuf, [fs + lanes], spread, mask=lanes < fn)
        return 0

    lax.fori_loop(0, 128, fillb, 0)

    def issue_or_drain(b, do_wait):
        len8 = _sget(len8_v, b)
        lsrc = _sget(loff8_v, b) * 8
        gdst = _sget(goff8_v, b) * 8
        nfull = _srl(len8, 4)

        def dma(src_sl, dst_sl):
            if do_wait:
                pltpu.make_async_copy(localbuf.at[src_sl],
                                      packedp_hbm.at[dst_sl], semS).wait()
            else:
                pltpu.async_copy(localbuf.at[src_sl],
                                 packedp_hbm.at[dst_sl], semS)

        def full(i, _):
            dma(pl.ds(lsrc + i * CH, CH), pl.ds(gdst + i * CH, CH))
            return 0

        lax.fori_loop(0, nfull, full, 0)
        o = nfull * CH
        rem = len8 & 15
        for k in (3, 2, 1, 0):
            n = 8 << k
            szk = lax.shift_right_logical(rem, k) & 1

            @pl.when(szk > 0)
            def _(o=o, n=n):
                dma(pl.ds(lsrc + o, n), pl.ds(gdst + o, n))

            o = o + szk * n

    def blk(bb, _):
        def ib(j, _):
            issue_or_drain(bb * 16 + j, False)
            return 0

        lax.fori_loop(0, 16, ib, 0)

        def db(j, _):
            issue_or_drain(bb * 16 + j, True)
            return 0

        lax.fori_loop(0, 16, db, 0)
        return 0

    lax.fori_loop(0, 8, blk, 0)

    for k in range(4):
        b = wid + k * NT

        @pl.when(b < NB)
        def _(k=k, b=b):
            gs = _sget(gapS_v, b)
            gn = _sget(gapN_v, b)
            for g in range(8):
                jvec = _lanes() + (g * 16)
                posbuf[pl.ds(g * 16, 16)] = jnp.where(
                    jvec < gn, gs + jvec, E_CAP + jvec
                )
                ebig[pl.ds(g * 16, 16)] = ((gs + jvec) & 0x7FFF) | (BK << 16)
            pltpu.sync_copy(ebig.at[pl.ds(0, CH)],
                            packedp_hbm.at[posbuf])


def _sc_permute(packed_e, counts):
    return pl.kernel(
        _perm_body,
        out_type=jax.ShapeDtypeStruct((E_ALL,), jnp.int32),
        mesh=plsc.VectorSubcoreMesh(**_MESH),
        **_CP,
        scratch_types=[
            pltpu.VMEM((NT, 128), jnp.int32),
            pltpu.VMEM((CH * SUP,), jnp.int32),
            pltpu.VMEM((EPT + 128 * 8,), jnp.int32),
            pltpu.VMEM((CH,), jnp.int32),
            pltpu.VMEM((128,), jnp.int32),
            pltpu.VMEM((128,), jnp.int32),
            pltpu.VMEM((128,), jnp.int32),
            pltpu.VMEM((128,), jnp.int32),
            pltpu.VMEM((128,), jnp.int32),
            pltpu.VMEM((128,), jnp.int32),
            pltpu.VMEM((128,), jnp.int32),
            pltpu.VMEM((128,), jnp.int32),
            pltpu.SemaphoreType.DMA,
        ],
    )(packed_e, counts)


def _deg_body(packedp_hbm, counts_hbm, deg_hbm, cntall, dbuf, starts_v,
              caps_v, slab):
    wid = _wid()
    pltpu.sync_copy(counts_hbm, cntall)
    _scan_counts(cntall, starts_v, caps_v)
    zf = jnp.zeros((16,), jnp.float32)

    for k in range(4):
        b = wid + k * NT

        @pl.when(b < NB)
        def _(b=b):
            def zs(i, _):
                slab[pl.ds(i * 16, 16)] = zf
                return 0

            lax.fori_loop(0, (BK + 32) // 16, zs, 0)
            st = _sget(starts_v, b) * CH
            nch = _sget(caps_v, b)

            def chunk(i, _):
                pltpu.sync_copy(packedp_hbm.at[pl.ds(st + i * CH, CH)], dbuf)

                def grp(g, _):
                    dvec = _srl(dbuf[pl.ds(g * 16, 16)], 16)
                    run, last = plsc.scan_count(dvec)
                    plsc.addupdate_scatter(slab, [dvec],
                                           run.astype(jnp.float32), mask=last)
                    return 0

                lax.fori_loop(0, CH // 16, grp, 0)
                return 0

            lax.fori_loop(0, nch, chunk, 0)

            def outg(g, _):
                slab[pl.ds(g * 16, 16)] = slab[pl.ds(g * 16, 16)] + 1.0
                return 0

            lax.fori_loop(0, BK // 16, outg, 0)
            pltpu.sync_copy(slab.at[pl.ds(0, BK)],
                            deg_hbm.at[pl.ds(b * BK, BK)])


def _sc_deg(packed_p, counts):
    return pl.kernel(
        _deg_body,
        out_type=jax.ShapeDtypeStruct((NPAD,), jnp.float32),
        mesh=plsc.VectorSubcoreMesh(**_MESH),
        **_CP,
        scratch_types=[
            pltpu.VMEM((NT, 128), jnp.int32),
            pltpu.VMEM((CH,), jnp.int32),
            pltpu.VMEM((128,), jnp.int32),
            pltpu.VMEM((128,), jnp.int32),
            pltpu.VMEM((BK + 32,), jnp.float32),
        ],
    )(packed_p, counts)


def _prop_body(packedp_hbm, counts_hbm, g_hbm, dinv_hbm, p_hbm,
               cntall, ebufA, ebufB, idxA, idxB, dlocA, dlocB, msgA, msgB,
               slab, dinvbuf, starts_v, caps_v, semA, semB, semW):
    wid = _wid()
    pltpu.sync_copy(counts_hbm, cntall)
    _scan_counts(cntall, starts_v, caps_v)
    zrow = jnp.zeros((16,), jnp.float32)

    def issue(st, c, eb, ib, db, mb, sm):
        pltpu.sync_copy(packedp_hbm.at[pl.ds(st + c * CH, CH)], eb)
        for g in range(8):
            cs = pl.ds(g * 16, 16)
            ev = eb[cs]
            ib[cs] = ev & MLOW
            db[cs] = _srl(ev, 16)
        pltpu.async_copy(g_hbm.at[ib], mb, sm)

    def drain(ib, mb, sm):
        pltpu.make_async_copy(g_hbm.at[ib], mb, sm).wait()

    def accum(db, mb):
        def grp(g, _):
            dvec = db[pl.ds(g * 16, 16)]
            for l0 in range(0, 16, 2):
                d0 = dvec[l0]
                d1 = dvec[l0 + 1]
                e0 = g * 16 + l0
                v0 = [mb[e0, pl.ds(j * 16, 16)] for j in range(8)]
                v1 = [mb[e0 + 1, pl.ds(j * 16, 16)] for j in range(8)]
                for j in range(8):
                    plsc.addupdate(slab.at[d0, pl.ds(j * 16, 16)], v0[j])
                for j in range(8):
                    plsc.addupdate(slab.at[d1, pl.ds(j * 16, 16)], v1[j])
            return 0

        lax.fori_loop(0, CH // 16, grp, 0)

    def bucket(k, _):
        b = wid + k * NT

        @pl.when(b < NB)
        def _():
            def zs(r, _):
                for j in range(8):
                    slab[r, pl.ds(j * 16, 16)] = zrow
                return 0

            lax.fori_loop(0, BK + 1, zs, 0)

            st = _sget(starts_v, b) * CH
            nch = _sget(caps_v, b)

            @pl.when(nch > 0)
            def _():
                issue(st, 0, ebufA, idxA, dlocA, msgA, semA)

                def pair(ip, _):
                    c0 = ip * 2

                    @pl.when(c0 + 1 < nch)
                    def _():
                        issue(st, c0 + 1, ebufB, idxB, dlocB, msgB, semB)

                    drain(idxA, msgA, semA)
                    accum(dlocA, msgA)

                    @pl.when(c0 + 2 < nch)
                    def _():
                        issue(st, c0 + 2, ebufA, idxA, dlocA, msgA, semA)

                    @pl.when(c0 + 1 < nch)
                    def _():
                        drain(idxB, msgB, semB)
                        accum(dlocB, msgB)

                    return 0

                lax.fori_loop(0, (nch + 1) // 2, pair, 0)

            pltpu.sync_copy(dinv_hbm.at[pl.ds(b * BK, BK)], dinvbuf)
            for c in range(4):
                buf = msgA if c % 2 == 0 else msgB
                if c >= 2:
                    prows = pl.ds(b * BK + (c - 2) * CH, CH)
                    pltpu.make_async_copy(buf, p_hbm.at[prows], semW).wait()

                def rgrp(g, _, c=c, buf=buf):
                    dvvec = dinvbuf[pl.ds(c * CH + g * 16, 16)]
                    for l in range(16):
                        r = g * 16 + l
                        lr = c * CH + r
                        dv = jnp.full((16,), dvvec[l], jnp.float32)
                        sv = [slab[lr, pl.ds(j * 16, 16)] for j in range(8)]
                        for j in range(8):
                            buf[r, pl.ds(j * 16, 16)] = sv[j] * dv
                    return 0

                lax.fori_loop(0, CH // 16, rgrp, 0)
                rows = pl.ds(b * BK + c * CH, CH)
                pltpu.async_copy(buf, p_hbm.at[rows], semW)
            for c in range(2, 4):
                buf = msgA if c % 2 == 0 else msgB
                rows = pl.ds(b * BK + c * CH, CH)
                pltpu.make_async_copy(buf, p_hbm.at[rows], semW).wait()

        return 0

    lax.fori_loop(0, 4, bucket, 0)


def _sc_prop(packed_p, counts, g, dinv):
    return pl.kernel(
        _prop_body,
        out_type=jax.ShapeDtypeStruct((NPAD, 128), jnp.float32),
        mesh=plsc.VectorSubcoreMesh(**_MESH),
        **_CP,
        scratch_types=[
            pltpu.VMEM((NT, 128), jnp.int32),
            pltpu.VMEM((CH,), jnp.int32),
            pltpu.VMEM((CH,), jnp.int32),
            pltpu.VMEM((CH,), jnp.int32),
            pltpu.VMEM((CH,), jnp.int32),
            pltpu.VMEM((CH,), jnp.int32),
            pltpu.VMEM((CH,), jnp.int32),
            pltpu.VMEM((CH, 128), jnp.float32),
            pltpu.VMEM((CH, 128), jnp.float32),
            pltpu.VMEM((BK + 1, 128), jnp.float32),
            pltpu.VMEM((BK,), jnp.float32),
            pltpu.VMEM((128,), jnp.int32),
            pltpu.VMEM((128,), jnp.int32),
            pltpu.SemaphoreType.DMA,
            pltpu.SemaphoreType.DMA,
            pltpu.SemaphoreType.DMA,
        ],
    )(packed_p, counts, g, dinv)


def _t1_body(deg_ref, x_ref, dinv_ref, g0_ref, q0_ref):
    dv = lax.rsqrt(deg_ref[...])
    g0 = x_ref[...] * dv
    dinv_ref[...] = dv
    g0_ref[...] = g0
    q0_ref[...] = g0 * dv


def _tc_stage1(deg2, x_pad):
    return pl.pallas_call(
        _t1_body,
        grid=(NB,),
        in_specs=[
            pl.BlockSpec((BK, 1), lambda i: (i, 0)),
            pl.BlockSpec((BK, 128), lambda i: (i, 0)),
        ],
        out_specs=[
            pl.BlockSpec((BK, 1), lambda i: (i, 0)),
            pl.BlockSpec((BK, 128), lambda i: (i, 0)),
            pl.BlockSpec((BK, 128), lambda i: (i, 0)),
        ],
        out_shape=[
            jax.ShapeDtypeStruct((NPAD, 1), jnp.float32),
            jax.ShapeDtypeStruct((NPAD, 128), jnp.float32),
            jax.ShapeDtypeStruct((NPAD, 128), jnp.float32),
        ],
    )(deg2, x_pad)


def _t2_body(ps_ref, q_ref, w_ref, b_ref, dinv_ref, g_ref, qo_ref):
    p = ps_ref[...] + q_ref[...]
    h = jnp.dot(p, w_ref[...], preferred_element_type=jnp.float32)
    h = jnp.maximum(h + b_ref[...], 0.0)
    dv = dinv_ref[...]
    g = h * dv
    g_ref[...] = g
    qo_ref[...] = g * dv


def _tc_layer(ps, q, w, bvec, dinv2):
    return pl.pallas_call(
        _t2_body,
        grid=(NB,),
        in_specs=[
            pl.BlockSpec((BK, 128), lambda i: (i, 0)),
            pl.BlockSpec((BK, 128), lambda i: (i, 0)),
            pl.BlockSpec((128, 128), lambda i: (0, 0)),
            pl.BlockSpec((1, 128), lambda i: (0, 0)),
            pl.BlockSpec((BK, 1), lambda i: (i, 0)),
        ],
        out_specs=[
            pl.BlockSpec((BK, 128), lambda i: (i, 0)),
            pl.BlockSpec((BK, 128), lambda i: (i, 0)),
        ],
        out_shape=[
            jax.ShapeDtypeStruct((NPAD, 128), jnp.float32),
            jax.ShapeDtypeStruct((NPAD, 128), jnp.float32),
        ],
    )(ps, q, w, bvec, dinv2)


def _t4_body(ps_ref, q_ref, w2_ref, b2_ref, wc_ref, bc_ref, out_ref):
    p = ps_ref[...] + q_ref[...]
    h = jnp.dot(p, w2_ref[...], preferred_element_type=jnp.float32)
    h = jnp.maximum(h + b2_ref[...], 0.0)
    out_ref[...] = (
        jnp.dot(h, wc_ref[...], preferred_element_type=jnp.float32)
        + bc_ref[...]
    )


def _tc_final(ps2, q2, w2, b2v, wcp, bcp):
    return pl.pallas_call(
        _t4_body,
        grid=(NB,),
        in_specs=[
            pl.BlockSpec((BK, 128), lambda i: (i, 0)),
            pl.BlockSpec((BK, 128), lambda i: (i, 0)),
            pl.BlockSpec((128, 128), lambda i: (0, 0)),
            pl.BlockSpec((1, 128), lambda i: (0, 0)),
            pl.BlockSpec((128, 8), lambda i: (0, 0)),
            pl.BlockSpec((1, 8), lambda i: (0, 0)),
        ],
        out_specs=pl.BlockSpec((BK, 8), lambda i: (i, 0)),
        out_shape=jax.ShapeDtypeStruct((NPAD, 8), jnp.float32),
    )(ps2, q2, w2, b2v, wcp, bcp)


def kernel(x, edge_index, W1, b1, W2, b2, Wc, bc):
    src_e = edge_index[0]
    dst_e = edge_index[1]
    x_pad = jnp.pad(x, ((0, NPAD - N), (0, 128 - x.shape[1])))
    W1p = jnp.pad(W1, ((0, 128 - W1.shape[0]), (0, 0)))
    Wcp = jnp.pad(Wc, ((0, 0), (0, 8 - Wc.shape[1])))
    b1r = b1.reshape(1, 128)
    b2r = b2.reshape(1, 128)
    bcp = jnp.pad(bc, (0, 8 - bc.shape[0])).reshape(1, 8)

    counts, packed_e = _sc_hist(src_e, dst_e)
    packed_p = _sc_permute(packed_e, counts)
    deg = _sc_deg(packed_p, counts)
    dinv2, g0, q0 = _tc_stage1(deg.reshape(NPAD, 1), x_pad)
    dinv = dinv2.reshape(NPAD)

    ps0 = _sc_prop(packed_p, counts, g0, dinv)
    g1, q1 = _tc_layer(ps0, q0, W1p, b1r, dinv2)
    ps1 = _sc_prop(packed_p, counts, g1, dinv)
    g2, q2 = _tc_layer(ps1, q1, W2, b2r, dinv2)
    ps2 = _sc_prop(packed_p, counts, g2, dinv)
    out = _tc_final(ps2, q2, W2, b2r, Wcp, bcp)
    return out[:N, :2]


# speedup vs baseline: 1.4867x; 1.0562x over previous
import jax
import jax.numpy as jnp
from jax import lax
from jax.experimental import pallas as pl
from jax.experimental.pallas import tpu as pltpu
from jax.experimental.pallas import tpu_sc as plsc

N = 50000
E = 800000
NB = 98
BK = 512
NPAD = NB * BK
NT = 32
EPT = E // NT
CH = 128
SUP = 13
NSUP = EPT // (CH * SUP)
TAIL = EPT - NSUP * CH * SUP
SENTB = 127
E_CAP = E + NB * (NT * 8 + CH)
E_ALL = E_CAP + CH
MLOW = 0xFFFF
MKEEP = 0x01FFFFFF

_MESH = dict(core_axis_name="c", subcore_axis_name="s")
_CP = dict(compiler_params=pltpu.CompilerParams(needs_layout_passes=False))


def _wid():
    return lax.axis_index("s") * 2 + lax.axis_index("c")


def _lanes():
    return lax.broadcasted_iota(jnp.int32, (16,), 0)


def _sget(ref, i):
    return plsc.load_gather(ref, [jnp.full((16,), i, jnp.int32)])[0]


def _srl(x, n):
    return lax.shift_right_logical(x, jnp.full(x.shape, n, jnp.int32))


def _hist_body(src_hbm, dst_hbm, counts_hbm, packed_hbm, sbuf, dbuf, pbuf,
               cnt):
    wid = _wid()
    base = wid * EPT
    z16 = jnp.zeros((16,), jnp.int32)
    for g in range(8):
        cnt[pl.ds(g * 16, 16)] = z16

    def count_group(bvec):
        run, last = plsc.scan_count(bvec)
        plsc.addupdate_scatter(cnt, [bvec], run, mask=last)

    def chunk(i, _):
        off = base + i * (CH * SUP)
        pltpu.sync_copy(src_hbm.at[pl.ds(off, CH * SUP)], sbuf)
        pltpu.sync_copy(dst_hbm.at[pl.ds(off, CH * SUP)], dbuf)

        def grp(g, _):
            cs = pl.ds(g * 16, 16)
            dv = dbuf[cs]
            count_group(_srl(dv, 9))
            pbuf[cs] = sbuf[cs] | lax.shift_left(dv, 16)
            return 0

        lax.fori_loop(0, CH * SUP // 16, grp, 0)
        pltpu.sync_copy(pbuf, packed_hbm.at[pl.ds(off, CH * SUP)])
        return 0

    lax.fori_loop(0, NSUP, chunk, 0)

    off = base + NSUP * CH * SUP
    pltpu.sync_copy(src_hbm.at[pl.ds(off, TAIL)], sbuf.at[pl.ds(0, TAIL)])
    pltpu.sync_copy(dst_hbm.at[pl.ds(off, TAIL)], dbuf.at[pl.ds(0, TAIL)])
    for g in range(3):
        cs = pl.ds(g * 16, 16)
        dv = dbuf[cs]
        bvec = _srl(dv, 9)
        if (g + 1) * 16 > TAIL:
            bvec = jnp.where(_lanes() < TAIL - g * 16, bvec, SENTB)
        count_group(bvec)
        pbuf[cs] = sbuf[cs] | lax.shift_left(dv, 16)
    pltpu.sync_copy(pbuf.at[pl.ds(0, TAIL)],
                    packed_hbm.at[pl.ds(off, TAIL)])

    pltpu.sync_copy(cnt, counts_hbm.at[wid])


def _sc_hist(src_e, dst_e):
    return pl.kernel(
        _hist_body,
        out_type=(
            jax.ShapeDtypeStruct((NT, 128), jnp.int32),
            jax.ShapeDtypeStruct((E,), jnp.int32),
        ),
        mesh=plsc.VectorSubcoreMesh(**_MESH),
        **_CP,
        scratch_types=[
            pltpu.VMEM((CH * SUP,), jnp.int32),
            pltpu.VMEM((CH * SUP,), jnp.int32),
            pltpu.VMEM((CH * SUP,), jnp.int32),
            pltpu.VMEM((128,), jnp.int32),
        ],
    )(src_e, dst_e)


def _scan_counts(cntall, starts_v, caps_v):
    carry = jnp.int32(0)
    for g in range(8):
        cs = pl.ds(g * 16, 16)

        def acc(t, tot):
            return tot + (cntall[t, cs] + 7) // 8 * 8

        tot8 = lax.fori_loop(0, NT, acc, jnp.zeros((16,), jnp.int32))
        capc = (tot8 + (CH - 1)) // CH
        cum = plsc.cumsum(capc)
        starts_v[cs] = cum - capc + carry
        caps_v[cs] = capc
        carry = carry + cum[15]


def _perm_body(packed_hbm, counts_hbm, packedp_hbm,
               cntall, ebig, localbuf, posbuf, goff8_v, loff8_v, len8_v,
               fillS_v, fillN_v, cursorL, gapS_v, gapN_v, semS):
    wid = _wid()
    pltpu.sync_copy(counts_hbm, cntall)

    carry = jnp.int32(0)
    lcarry = jnp.int32(0)
    z16 = jnp.zeros((16,), jnp.int32)
    for g in range(8):
        cs = pl.ds(g * 16, 16)

        def acc(t, tm):
            tot8, mine8 = tm
            v8 = (cntall[t, cs] + 7) // 8 * 8
            return tot8 + v8, mine8 + jnp.where(t < wid, v8, 0)

        tot8, mine8 = lax.fori_loop(0, NT, acc, (z16, z16))
        mycnt = cntall[wid, cs]
        myr8 = (mycnt + 7) // 8 * 8
        cap = (tot8 + (CH - 1)) // CH * CH
        cum = plsc.cumsum(cap)
        gstart = cum - cap + carry
        goff8_v[cs] = _srl(gstart + mine8, 3)
        gapS_v[cs] = gstart + tot8
        gapN_v[cs] = cap - tot8
        carry = carry + cum[15]

        lcum = plsc.cumsum(myr8)
        loff = lcum - myr8 + lcarry
        loff8_v[cs] = _srl(loff, 3)
        len8_v[cs] = _srl(myr8, 3)
        fillS_v[cs] = loff + mycnt
        fillN_v[cs] = myr8 - mycnt
        cursorL[cs] = loff
        lcarry = lcarry + lcum[15]

    def place_group(bvec, vals):
        run, last = plsc.scan_count(bvec)
        basev = plsc.load_gather(cursorL, [bvec])
        plsc.store_scatter(localbuf, [basev + run - 1], vals)
        plsc.addupdate_scatter(cursorL, [bvec], run, mask=last)

    base = wid * EPT

    def superchunk(sc, _):
        off = base + sc * (CH * SUP)
        pltpu.sync_copy(packed_hbm.at[pl.ds(off, CH * SUP)], ebig)

        def grp(g, _):
            cs = pl.ds(g * 16, 16)
            ev = ebig[cs]
            place_group(_srl(ev, 25), ev & MKEEP)
            return 0

        lax.fori_loop(0, CH * SUP // 16, grp, 0)
        return 0

    lax.fori_loop(0, NSUP, superchunk, 0)

    off = base + NSUP * CH * SUP
    pltpu.sync_copy(packed_hbm.at[pl.ds(off, TAIL)], ebig.at[pl.ds(0, TAIL)])
    for g in range(3):
        ev = ebig[pl.ds(g * 16, 16)]
        bvec = _srl(ev, 25)
        if (g + 1) * 16 > TAIL:
            bvec = jnp.where(_lanes() < TAIL - g * 16, bvec, SENTB)
        place_group(bvec, ev & MKEEP)

    lanes = _lanes()

    def fillb(b, _):
        fs = _sget(fillS_v, b)
        fn = _sget(fillN_v, b)
        spread = ((fs + lanes + wid * 953) & 0x7FFF) | (BK << 16)
        plsc.store_scatter(localbuf, [fs + lanes], spread, mask=lanes < fn)
        return 0

    lax.fori_loop(0, 128, fillb, 0)

    def issue_or_drain(b, do_wait):
        len8 = _sget(len8_v, b)
        lsrc = _sget(loff8_v, b) * 8
        gdst = _sget(goff8_v, b) * 8
        nfull = _srl(len8, 4)

        def dma(src_sl, dst_sl):
            if do_wait:
                pltpu.make_async_copy(localbuf.at[src_sl],
                                      packedp_hbm.at[dst_sl], semS).wait()
            else:
                pltpu.async_copy(localbuf.at[src_sl],
                                 packedp_hbm.at[dst_sl], semS)

        def full(i, _):
            dma(pl.ds(lsrc + i * CH, CH), pl.ds(gdst + i * CH, CH))
            return 0

        lax.fori_loop(0, nfull, full, 0)
        o = nfull * CH
        rem = len8 & 15
        for k in (3, 2, 1, 0):
            n = 8 << k
            szk = lax.shift_right_logical(rem, k) & 1

            @pl.when(szk > 0)
            def _(o=o, n=n):
                dma(pl.ds(lsrc + o, n), pl.ds(gdst + o, n))

            o = o + szk * n

    def ib(j, _):
        issue_or_drain(j, False)
        return 0

    def blk(bb, _):
        def ib2(j, _, ):
            issue_or_drain(bb * 16 + j, False)
            return 0

        lax.fori_loop(0, 16, ib2, 0)

        def db(j, _):
            issue_or_drain((bb - 1) * 16 + j, True)
            return 0

        lax.fori_loop(0, 16, db, 0)
        return 0

    lax.fori_loop(0, 16, ib, 0)
    lax.fori_loop(1, 8, blk, 0)

    def dbl(j, _):
        issue_or_drain(112 + j, True)
        return 0

    lax.fori_loop(0, 16, dbl, 0)

    for k in range(4):
        b = wid + k * NT

        @pl.when(b < NB)
        def _(k=k, b=b):
            gs = _sget(gapS_v, b)
            gn = _sget(gapN_v, b)
            for g in range(8):
                jvec = _lanes() + (g * 16)
                posbuf[pl.ds(g * 16, 16)] = jnp.where(
                    jvec < gn, gs + jvec, E_CAP + jvec
                )
                ebig[pl.ds(g * 16, 16)] = ((gs + jvec) & 0x7FFF) | (BK << 16)
            pltpu.sync_copy(ebig.at[pl.ds(0, CH)],
                            packedp_hbm.at[posbuf])


def _sc_permute(packed_e, counts):
    return pl.kernel(
        _perm_body,
        out_type=jax.ShapeDtypeStruct((E_ALL,), jnp.int32),
        mesh=plsc.VectorSubcoreMesh(**_MESH),
        **_CP,
        scratch_types=[
            pltpu.VMEM((NT, 128), jnp.int32),
            pltpu.VMEM((CH * SUP,), jnp.int32),
            pltpu.VMEM((EPT + 128 * 8,), jnp.int32),
            pltpu.VMEM((CH,), jnp.int32),
            pltpu.VMEM((128,), jnp.int32),
            pltpu.VMEM((128,), jnp.int32),
            pltpu.VMEM((128,), jnp.int32),
            pltpu.VMEM((128,), jnp.int32),
            pltpu.VMEM((128,), jnp.int32),
            pltpu.VMEM((128,), jnp.int32),
            pltpu.VMEM((128,), jnp.int32),
            pltpu.VMEM((128,), jnp.int32),
            pltpu.SemaphoreType.DMA,
        ],
    )(packed_e, counts)


def _deg_body(packedp_hbm, counts_hbm, deg_hbm, cntall, dbuf, starts_v,
              caps_v, slab):
    wid = _wid()
    pltpu.sync_copy(counts_hbm, cntall)
    _scan_counts(cntall, starts_v, caps_v)
    zf = jnp.zeros((16,), jnp.float32)

    for k in range(4):
        b = wid + k * NT

        @pl.when(b < NB)
        def _(b=b):
            def zs(i, _):
                slab[pl.ds(i * 16, 16)] = zf
                return 0

            lax.fori_loop(0, (BK + 32) // 16, zs, 0)
            st = _sget(starts_v, b) * CH
            nch = _sget(caps_v, b)

            def chunk(i, _):
                pltpu.sync_copy(packedp_hbm.at[pl.ds(st + i * CH, CH)], dbuf)

                def grp(g, _):
                    dvec = _srl(dbuf[pl.ds(g * 16, 16)], 16)
                    run, last = plsc.scan_count(dvec)
                    plsc.addupdate_scatter(slab, [dvec],
                                           run.astype(jnp.float32), mask=last)
                    return 0

                lax.fori_loop(0, CH // 16, grp, 0)
                return 0

            lax.fori_loop(0, nch, chunk, 0)

            def outg(g, _):
                slab[pl.ds(g * 16, 16)] = slab[pl.ds(g * 16, 16)] + 1.0
                return 0

            lax.fori_loop(0, BK // 16, outg, 0)
            pltpu.sync_copy(slab.at[pl.ds(0, BK)],
                            deg_hbm.at[pl.ds(b * BK, BK)])


def _sc_deg(packed_p, counts):
    return pl.kernel(
        _deg_body,
        out_type=jax.ShapeDtypeStruct((NPAD,), jnp.float32),
        mesh=plsc.VectorSubcoreMesh(**_MESH),
        **_CP,
        scratch_types=[
            pltpu.VMEM((NT, 128), jnp.int32),
            pltpu.VMEM((CH,), jnp.int32),
            pltpu.VMEM((128,), jnp.int32),
            pltpu.VMEM((128,), jnp.int32),
            pltpu.VMEM((BK + 32,), jnp.float32),
        ],
    )(packed_p, counts)


def _prop_body(packedp_hbm, counts_hbm, g_hbm, dinv_hbm, p_hbm,
               cntall, ebufA, ebufB, idxA, idxB, dlocA, dlocB, msgA, msgB,
               slab, dinvbuf, starts_v, caps_v, semA, semB, semW):
    wid = _wid()
    pltpu.sync_copy(counts_hbm, cntall)
    _scan_counts(cntall, starts_v, caps_v)
    zrow = jnp.zeros((16,), jnp.float32)

    def issue(st, c, eb, ib, db, mb, sm):
        pltpu.sync_copy(packedp_hbm.at[pl.ds(st + c * CH, CH)], eb)
        for g in range(8):
            cs = pl.ds(g * 16, 16)
            ev = eb[cs]
            ib[cs] = ev & MLOW
            db[cs] = _srl(ev, 16)
        pltpu.async_copy(g_hbm.at[ib], mb, sm)

    def drain(ib, mb, sm):
        pltpu.make_async_copy(g_hbm.at[ib], mb, sm).wait()

    def accum(db, mb):
        def grp(g, _):
            dvec = db[pl.ds(g * 16, 16)]
            for l0 in range(0, 16, 2):
                d0 = dvec[l0]
                d1 = dvec[l0 + 1]
                e0 = g * 16 + l0
                v0 = [mb[e0, pl.ds(j * 16, 16)] for j in range(8)]
                v1 = [mb[e0 + 1, pl.ds(j * 16, 16)] for j in range(8)]
                for j in range(8):
                    plsc.addupdate(slab.at[d0, pl.ds(j * 16, 16)], v0[j])
                for j in range(8):
                    plsc.addupdate(slab.at[d1, pl.ds(j * 16, 16)], v1[j])
            return 0

        lax.fori_loop(0, CH // 16, grp, 0)

    def bucket(k, _):
        b = wid + k * NT

        @pl.when(b < NB)
        def _():
            def zs(r, _):
                for j in range(8):
                    slab[r, pl.ds(j * 16, 16)] = zrow
                return 0

            lax.fori_loop(0, BK + 1, zs, 0)

            st = _sget(starts_v, b) * CH
            nch = _sget(caps_v, b)

            @pl.when(nch > 0)
            def _():
                issue(st, 0, ebufA, idxA, dlocA, msgA, semA)

                def pair(ip, _):
                    c0 = ip * 2

                    @pl.when(c0 + 1 < nch)
                    def _():
                        issue(st, c0 + 1, ebufB, idxB, dlocB, msgB, semB)

                    drain(idxA, msgA, semA)
                    accum(dlocA, msgA)

                    @pl.when(c0 + 2 < nch)
                    def _():
                        issue(st, c0 + 2, ebufA, idxA, dlocA, msgA, semA)

                    @pl.when(c0 + 1 < nch)
                    def _():
                        drain(idxB, msgB, semB)
                        accum(dlocB, msgB)

                    return 0

                lax.fori_loop(0, (nch + 1) // 2, pair, 0)

            pltpu.sync_copy(dinv_hbm.at[pl.ds(b * BK, BK)], dinvbuf)
            for c in range(4):
                buf = msgA if c % 2 == 0 else msgB
                if c >= 2:
                    prows = pl.ds(b * BK + (c - 2) * CH, CH)
                    pltpu.make_async_copy(buf, p_hbm.at[prows], semW).wait()

                def rgrp(g, _, c=c, buf=buf):
                    dvvec = dinvbuf[pl.ds(c * CH + g * 16, 16)]
                    for l in range(16):
                        r = g * 16 + l
                        lr = c * CH + r
                        dv = jnp.full((16,), dvvec[l], jnp.float32)
                        sv = [slab[lr, pl.ds(j * 16, 16)] for j in range(8)]
                        for j in range(8):
                            buf[r, pl.ds(j * 16, 16)] = sv[j] * dv
                    return 0

                lax.fori_loop(0, CH // 16, rgrp, 0)
                rows = pl.ds(b * BK + c * CH, CH)
                pltpu.async_copy(buf, p_hbm.at[rows], semW)
            for c in range(2, 4):
                buf = msgA if c % 2 == 0 else msgB
                rows = pl.ds(b * BK + c * CH, CH)
                pltpu.make_async_copy(buf, p_hbm.at[rows], semW).wait()

        return 0

    lax.fori_loop(0, 4, bucket, 0)


def _sc_prop(packed_p, counts, g, dinv):
    return pl.kernel(
        _prop_body,
        out_type=jax.ShapeDtypeStruct((NPAD, 128), jnp.float32),
        mesh=plsc.VectorSubcoreMesh(**_MESH),
        **_CP,
        scratch_types=[
            pltpu.VMEM((NT, 128), jnp.int32),
            pltpu.VMEM((CH,), jnp.int32),
            pltpu.VMEM((CH,), jnp.int32),
            pltpu.VMEM((CH,), jnp.int32),
            pltpu.VMEM((CH,), jnp.int32),
            pltpu.VMEM((CH,), jnp.int32),
            pltpu.VMEM((CH,), jnp.int32),
            pltpu.VMEM((CH, 128), jnp.float32),
            pltpu.VMEM((CH, 128), jnp.float32),
            pltpu.VMEM((BK + 1, 128), jnp.float32),
            pltpu.VMEM((BK,), jnp.float32),
            pltpu.VMEM((128,), jnp.int32),
            pltpu.VMEM((128,), jnp.int32),
            pltpu.SemaphoreType.DMA,
            pltpu.SemaphoreType.DMA,
            pltpu.SemaphoreType.DMA,
        ],
    )(packed_p, counts, g, dinv)


def _t1_body(deg_ref, x_ref, dinv_ref, g0_ref, q0_ref):
    dv = lax.rsqrt(deg_ref[...])
    g0 = x_ref[...] * dv
    dinv_ref[...] = dv
    g0_ref[...] = g0
    q0_ref[...] = g0 * dv


def _tc_stage1(deg2, x_pad):
    return pl.pallas_call(
        _t1_body,
        grid=(NB,),
        in_specs=[
            pl.BlockSpec((BK, 1), lambda i: (i, 0)),
            pl.BlockSpec((BK, 128), lambda i: (i, 0)),
        ],
        out_specs=[
            pl.BlockSpec((BK, 1), lambda i: (i, 0)),
            pl.BlockSpec((BK, 128), lambda i: (i, 0)),
            pl.BlockSpec((BK, 128), lambda i: (i, 0)),
        ],
        out_shape=[
            jax.ShapeDtypeStruct((NPAD, 1), jnp.float32),
            jax.ShapeDtypeStruct((NPAD, 128), jnp.float32),
            jax.ShapeDtypeStruct((NPAD, 128), jnp.float32),
        ],
    )(deg2, x_pad)


def _t2_body(ps_ref, q_ref, w_ref, b_ref, dinv_ref, g_ref, qo_ref):
    p = ps_ref[...] + q_ref[...]
    h = jnp.dot(p, w_ref[...], preferred_element_type=jnp.float32)
    h = jnp.maximum(h + b_ref[...], 0.0)
    dv = dinv_ref[...]
    g = h * dv
    g_ref[...] = g
    qo_ref[...] = g * dv


def _tc_layer(ps, q, w, bvec, dinv2):
    return pl.pallas_call(
        _t2_body,
        grid=(NB,),
        in_specs=[
            pl.BlockSpec((BK, 128), lambda i: (i, 0)),
            pl.BlockSpec((BK, 128), lambda i: (i, 0)),
            pl.BlockSpec((128, 128), lambda i: (0, 0)),
            pl.BlockSpec((1, 128), lambda i: (0, 0)),
            pl.BlockSpec((BK, 1), lambda i: (i, 0)),
        ],
        out_specs=[
            pl.BlockSpec((BK, 128), lambda i: (i, 0)),
            pl.BlockSpec((BK, 128), lambda i: (i, 0)),
        ],
        out_shape=[
            jax.ShapeDtypeStruct((NPAD, 128), jnp.float32),
            jax.ShapeDtypeStruct((NPAD, 128), jnp.float32),
        ],
    )(ps, q, w, bvec, dinv2)


def _t4_body(ps_ref, q_ref, w2_ref, b2_ref, wc_ref, bc_ref, out_ref):
    p = ps_ref[...] + q_ref[...]
    h = jnp.dot(p, w2_ref[...], preferred_element_type=jnp.float32)
    h = jnp.maximum(h + b2_ref[...], 0.0)
    out_ref[...] = (
        jnp.dot(h, wc_ref[...], preferred_element_type=jnp.float32)
        + bc_ref[...]
    )


def _tc_final(ps2, q2, w2, b2v, wcp, bcp):
    return pl.pallas_call(
        _t4_body,
        grid=(NB,),
        in_specs=[
            pl.BlockSpec((BK, 128), lambda i: (i, 0)),
            pl.BlockSpec((BK, 128), lambda i: (i, 0)),
            pl.BlockSpec((128, 128), lambda i: (0, 0)),
            pl.BlockSpec((1, 128), lambda i: (0, 0)),
            pl.BlockSpec((128, 8), lambda i: (0, 0)),
            pl.BlockSpec((1, 8), lambda i: (0, 0)),
        ],
        out_specs=pl.BlockSpec((BK, 8), lambda i: (i, 0)),
        out_shape=jax.ShapeDtypeStruct((NPAD, 8), jnp.float32),
    )(ps2, q2, w2, b2v, wcp, bcp)


def kernel(x, edge_index, W1, b1, W2, b2, Wc, bc):
    src_e = edge_index[0]
    dst_e = edge_index[1]
    x_pad = jnp.pad(x, ((0, NPAD - N), (0, 128 - x.shape[1])))
    W1p = jnp.pad(W1, ((0, 128 - W1.shape[0]), (0, 0)))
    Wcp = jnp.pad(Wc, ((0, 0), (0, 8 - Wc.shape[1])))
    b1r = b1.reshape(1, 128)
    b2r = b2.reshape(1, 128)
    bcp = jnp.pad(bc, (0, 8 - bc.shape[0])).reshape(1, 8)

    counts, packed_e = _sc_hist(src_e, dst_e)
    packed_p = _sc_permute(packed_e, counts)
    deg = _sc_deg(packed_p, counts)
    dinv2, g0, q0 = _tc_stage1(deg.reshape(NPAD, 1), x_pad)
    dinv = dinv2.reshape(NPAD)

    ps0 = _sc_prop(packed_p, counts, g0, dinv)
    g1, q1 = _tc_layer(ps0, q0, W1p, b1r, dinv2)
    ps1 = _sc_prop(packed_p, counts, g1, dinv)
    g2, q2 = _tc_layer(ps1, q1, W2, b2r, dinv2)
    ps2 = _sc_prop(packed_p, counts, g2, dinv)
    out = _tc_final(ps2, q2, W2, b2r, Wcp, bcp)
    return out[:N, :2]


# speedup vs baseline: 1.4977x; 1.0074x over previous
import jax
import jax.numpy as jnp
from jax import lax
from jax.experimental import pallas as pl
from jax.experimental.pallas import tpu as pltpu
from jax.experimental.pallas import tpu_sc as plsc

N = 50000
E = 800000
NB = 98
BK = 512
NPAD = NB * BK
NT = 32
EPT = E // NT
CH = 128
SUP = 13
NSUP = EPT // (CH * SUP)
TAIL = EPT - NSUP * CH * SUP
SENTB = 127
E_CAP = E + NB * (NT * 8 + CH)
E_ALL = E_CAP + CH
MLOW = 0xFFFF
MKEEP = 0x01FFFFFF

_MESH = dict(core_axis_name="c", subcore_axis_name="s")
_CP = dict(compiler_params=pltpu.CompilerParams(needs_layout_passes=False))


def _wid():
    return lax.axis_index("s") * 2 + lax.axis_index("c")


def _lanes():
    return lax.broadcasted_iota(jnp.int32, (16,), 0)


def _sget(ref, i):
    return plsc.load_gather(ref, [jnp.full((16,), i, jnp.int32)])[0]


def _srl(x, n):
    return lax.shift_right_logical(x, jnp.full(x.shape, n, jnp.int32))


def _hist_body(src_hbm, dst_hbm, counts_hbm, packed_hbm, sbuf, dbuf, pbuf,
               cnt):
    wid = _wid()
    base = wid * EPT
    z16 = jnp.zeros((16,), jnp.int32)
    for g in range(8):
        cnt[pl.ds(g * 16, 16)] = z16

    def count_group(bvec):
        run, last = plsc.scan_count(bvec)
        plsc.addupdate_scatter(cnt, [bvec], run, mask=last)

    def chunk(i, _):
        off = base + i * (CH * SUP)
        pltpu.sync_copy(src_hbm.at[pl.ds(off, CH * SUP)], sbuf)
        pltpu.sync_copy(dst_hbm.at[pl.ds(off, CH * SUP)], dbuf)

        def grp(g, _):
            cs = pl.ds(g * 16, 16)
            dv = dbuf[cs]
            count_group(_srl(dv, 9))
            pbuf[cs] = sbuf[cs] | lax.shift_left(dv, 16)
            return 0

        lax.fori_loop(0, CH * SUP // 16, grp, 0)
        pltpu.sync_copy(pbuf, packed_hbm.at[pl.ds(off, CH * SUP)])
        return 0

    lax.fori_loop(0, NSUP, chunk, 0)

    off = base + NSUP * CH * SUP
    pltpu.sync_copy(src_hbm.at[pl.ds(off, TAIL)], sbuf.at[pl.ds(0, TAIL)])
    pltpu.sync_copy(dst_hbm.at[pl.ds(off, TAIL)], dbuf.at[pl.ds(0, TAIL)])
    for g in range(3):
        cs = pl.ds(g * 16, 16)
        dv = dbuf[cs]
        bvec = _srl(dv, 9)
        if (g + 1) * 16 > TAIL:
            bvec = jnp.where(_lanes() < TAIL - g * 16, bvec, SENTB)
        count_group(bvec)
        pbuf[cs] = sbuf[cs] | lax.shift_left(dv, 16)
    pltpu.sync_copy(pbuf.at[pl.ds(0, TAIL)],
                    packed_hbm.at[pl.ds(off, TAIL)])

    pltpu.sync_copy(cnt, counts_hbm.at[wid])


def _sc_hist(src_e, dst_e):
    return pl.kernel(
        _hist_body,
        out_type=(
            jax.ShapeDtypeStruct((NT, 128), jnp.int32),
            jax.ShapeDtypeStruct((E,), jnp.int32),
        ),
        mesh=plsc.VectorSubcoreMesh(**_MESH),
        **_CP,
        scratch_types=[
            pltpu.VMEM((CH * SUP,), jnp.int32),
            pltpu.VMEM((CH * SUP,), jnp.int32),
            pltpu.VMEM((CH * SUP,), jnp.int32),
            pltpu.VMEM((128,), jnp.int32),
        ],
    )(src_e, dst_e)


def _scan_counts(cntall, starts_v, caps_v):
    carry = jnp.int32(0)
    for g in range(8):
        cs = pl.ds(g * 16, 16)

        def acc(t, tot):
            return tot + (cntall[t, cs] + 7) // 8 * 8

        tot8 = lax.fori_loop(0, NT, acc, jnp.zeros((16,), jnp.int32))
        capc = (tot8 + (CH - 1)) // CH
        cum = plsc.cumsum(capc)
        starts_v[cs] = cum - capc + carry
        caps_v[cs] = capc
        carry = carry + cum[15]


def _perm_body(packed_hbm, counts_hbm, packedp_hbm,
               cntall, ebig, localbuf, posbuf, goff8_v, loff8_v, len8_v,
               fillS_v, fillN_v, cursorL, gapS_v, gapN_v, semS):
    wid = _wid()
    pltpu.sync_copy(counts_hbm, cntall)

    carry = jnp.int32(0)
    lcarry = jnp.int32(0)
    z16 = jnp.zeros((16,), jnp.int32)
    for g in range(8):
        cs = pl.ds(g * 16, 16)

        def acc(t, tm):
            tot8, mine8 = tm
            v8 = (cntall[t, cs] + 7) // 8 * 8
            return tot8 + v8, mine8 + jnp.where(t < wid, v8, 0)

        tot8, mine8 = lax.fori_loop(0, NT, acc, (z16, z16))
        mycnt = cntall[wid, cs]
        myr8 = (mycnt + 7) // 8 * 8
        cap = (tot8 + (CH - 1)) // CH * CH
        cum = plsc.cumsum(cap)
        gstart = cum - cap + carry
        goff8_v[cs] = _srl(gstart + mine8, 3)
        gapS_v[cs] = gstart + tot8
        gapN_v[cs] = cap - tot8
        carry = carry + cum[15]

        lcum = plsc.cumsum(myr8)
        loff = lcum - myr8 + lcarry
        loff8_v[cs] = _srl(loff, 3)
        len8_v[cs] = _srl(myr8, 3)
        fillS_v[cs] = loff + mycnt
        fillN_v[cs] = myr8 - mycnt
        cursorL[cs] = loff
        lcarry = lcarry + lcum[15]

    def place_group(bvec, vals):
        run, last = plsc.scan_count(bvec)
        basev = plsc.load_gather(cursorL, [bvec])
        plsc.store_scatter(localbuf, [basev + run - 1], vals)
        plsc.addupdate_scatter(cursorL, [bvec], run, mask=last)

    base = wid * EPT

    def superchunk(sc, _):
        off = base + sc * (CH * SUP)
        pltpu.sync_copy(packed_hbm.at[pl.ds(off, CH * SUP)], ebig)

        def grp(g, _):
            cs = pl.ds(g * 16, 16)
            ev = ebig[cs]
            place_group(_srl(ev, 25), ev & MKEEP)
            return 0

        lax.fori_loop(0, CH * SUP // 16, grp, 0)
        return 0

    lax.fori_loop(0, NSUP, superchunk, 0)

    off = base + NSUP * CH * SUP
    pltpu.sync_copy(packed_hbm.at[pl.ds(off, TAIL)], ebig.at[pl.ds(0, TAIL)])
    for g in range(3):
        ev = ebig[pl.ds(g * 16, 16)]
        bvec = _srl(ev, 25)
        if (g + 1) * 16 > TAIL:
            bvec = jnp.where(_lanes() < TAIL - g * 16, bvec, SENTB)
        place_group(bvec, ev & MKEEP)

    lanes = _lanes()

    def fillb(b, _):
        fs = _sget(fillS_v, b)
        fn = _sget(fillN_v, b)
        spread = ((fs + lanes + wid * 953) & 0x7FFF) | (BK << 16)
        plsc.store_scatter(localbuf, [fs + lanes], spread, mask=lanes < fn)
        return 0

    lax.fori_loop(0, 128, fillb, 0)

    def issue_or_drain(b, do_wait):
        len8 = _sget(len8_v, b)
        lsrc = _sget(loff8_v, b) * 8
        gdst = _sget(goff8_v, b) * 8
        nfull = _srl(len8, 4)

        def dma(src_sl, dst_sl):
            if do_wait:
                pltpu.make_async_copy(localbuf.at[src_sl],
                                      packedp_hbm.at[dst_sl], semS).wait()
            else:
                pltpu.async_copy(localbuf.at[src_sl],
                                 packedp_hbm.at[dst_sl], semS)

        def full(i, _):
            dma(pl.ds(lsrc + i * CH, CH), pl.ds(gdst + i * CH, CH))
            return 0

        lax.fori_loop(0, nfull, full, 0)
        o = nfull * CH
        rem = len8 & 15
        for k in (3, 2, 1, 0):
            n = 8 << k
            szk = lax.shift_right_logical(rem, k) & 1

            @pl.when(szk > 0)
            def _(o=o, n=n):
                dma(pl.ds(lsrc + o, n), pl.ds(gdst + o, n))

            o = o + szk * n

    def ib(j, _):
        issue_or_drain(j, False)
        return 0

    def blk(bb, _):
        def ib2(j, _, ):
            issue_or_drain(bb * 16 + j, False)
            return 0

        lax.fori_loop(0, 16, ib2, 0)

        def db(j, _):
            issue_or_drain((bb - 1) * 16 + j, True)
            return 0

        lax.fori_loop(0, 16, db, 0)
        return 0

    lax.fori_loop(0, 16, ib, 0)
    lax.fori_loop(1, 8, blk, 0)

    def dbl(j, _):
        issue_or_drain(112 + j, True)
        return 0

    lax.fori_loop(0, 16, dbl, 0)

    for k in range(4):
        b = wid + k * NT

        @pl.when(b < NB)
        def _(k=k, b=b):
            gs = _sget(gapS_v, b)
            gn = _sget(gapN_v, b)
            for g in range(8):
                jvec = _lanes() + (g * 16)
                posbuf[pl.ds(g * 16, 16)] = jnp.where(
                    jvec < gn, gs + jvec, E_CAP + jvec
                )
                ebig[pl.ds(g * 16, 16)] = ((gs + jvec) & 0x7FFF) | (BK << 16)
            pltpu.sync_copy(ebig.at[pl.ds(0, CH)],
                            packedp_hbm.at[posbuf])


def _sc_permute(packed_e, counts):
    return pl.kernel(
        _perm_body,
        out_type=jax.ShapeDtypeStruct((E_ALL,), jnp.int32),
        mesh=plsc.VectorSubcoreMesh(**_MESH),
        **_CP,
        scratch_types=[
            pltpu.VMEM((NT, 128), jnp.int32),
            pltpu.VMEM((CH * SUP,), jnp.int32),
            pltpu.VMEM((EPT + 128 * 8,), jnp.int32),
            pltpu.VMEM((CH,), jnp.int32),
            pltpu.VMEM((128,), jnp.int32),
            pltpu.VMEM((128,), jnp.int32),
            pltpu.VMEM((128,), jnp.int32),
            pltpu.VMEM((128,), jnp.int32),
            pltpu.VMEM((128,), jnp.int32),
            pltpu.VMEM((128,), jnp.int32),
            pltpu.VMEM((128,), jnp.int32),
            pltpu.VMEM((128,), jnp.int32),
            pltpu.SemaphoreType.DMA,
        ],
    )(packed_e, counts)


def _deg_body(packedp_hbm, counts_hbm, deg_hbm, cntall, dbuf, starts_v,
              caps_v, slab):
    wid = _wid()
    pltpu.sync_copy(counts_hbm, cntall)
    _scan_counts(cntall, starts_v, caps_v)
    zf = jnp.zeros((16,), jnp.float32)

    for k in range(4):
        b = wid + k * NT

        @pl.when(b < NB)
        def _(b=b):
            def zs(i, _):
                slab[pl.ds(i * 16, 16)] = zf
                return 0

            lax.fori_loop(0, (BK + 32) // 16, zs, 0)
            st = _sget(starts_v, b) * CH
            nch = _sget(caps_v, b)

            def chunk(i, _):
                pltpu.sync_copy(packedp_hbm.at[pl.ds(st + i * CH, CH)], dbuf)

                def grp(g, _):
                    dvec = _srl(dbuf[pl.ds(g * 16, 16)], 16)
                    run, last = plsc.scan_count(dvec)
                    plsc.addupdate_scatter(slab, [dvec],
                                           run.astype(jnp.float32), mask=last)
                    return 0

                lax.fori_loop(0, CH // 16, grp, 0)
                return 0

            lax.fori_loop(0, nch, chunk, 0)

            def outg(g, _):
                slab[pl.ds(g * 16, 16)] = slab[pl.ds(g * 16, 16)] + 1.0
                return 0

            lax.fori_loop(0, BK // 16, outg, 0)
            pltpu.sync_copy(slab.at[pl.ds(0, BK)],
                            deg_hbm.at[pl.ds(b * BK, BK)])


def _sc_deg(packed_p, counts):
    return pl.kernel(
        _deg_body,
        out_type=jax.ShapeDtypeStruct((NPAD,), jnp.float32),
        mesh=plsc.VectorSubcoreMesh(**_MESH),
        **_CP,
        scratch_types=[
            pltpu.VMEM((NT, 128), jnp.int32),
            pltpu.VMEM((CH,), jnp.int32),
            pltpu.VMEM((128,), jnp.int32),
            pltpu.VMEM((128,), jnp.int32),
            pltpu.VMEM((BK + 32,), jnp.float32),
        ],
    )(packed_p, counts)


def _prop_body(packedp_hbm, counts_hbm, g_hbm, dinv_hbm, p_hbm,
               cntall, ebufA, ebufB, idxA, idxB, dlocA, dlocB, msgA, msgB,
               slab, dinvbuf, starts_v, caps_v, semA, semB, semW):
    wid = _wid()
    pltpu.sync_copy(counts_hbm, cntall)
    _scan_counts(cntall, starts_v, caps_v)
    zrow = jnp.zeros((16,), jnp.float32)

    def issue(st, c, eb, ib, db, mb, sm):
        pltpu.sync_copy(packedp_hbm.at[pl.ds(st + c * CH, CH)], eb)
        for g in range(8):
            cs = pl.ds(g * 16, 16)
            ev = eb[cs]
            ib[cs] = ev & MLOW
            db[cs] = _srl(ev, 16)
        pltpu.async_copy(g_hbm.at[ib], mb, sm)

    def drain(ib, mb, sm):
        pltpu.make_async_copy(g_hbm.at[ib], mb, sm).wait()

    def accum(db, mb):
        def grp(g, _):
            dvec = db[pl.ds(g * 16, 16)]
            for l0 in range(0, 16, 4):
                ds_ = [dvec[l0 + i] for i in range(4)]
                es = [g * 16 + l0 + i for i in range(4)]
                vs = [[mb[e, pl.ds(j * 16, 16)] for j in range(8)]
                      for e in es]
                for i in range(4):
                    for j in range(8):
                        plsc.addupdate(slab.at[ds_[i], pl.ds(j * 16, 16)],
                                       vs[i][j])
            return 0

        lax.fori_loop(0, CH // 16, grp, 0)

    def bucket(k, _):
        b = wid + k * NT

        @pl.when(b < NB)
        def _():
            def zs(r, _):
                for j in range(8):
                    slab[r, pl.ds(j * 16, 16)] = zrow
                return 0

            lax.fori_loop(0, BK + 1, zs, 0)

            st = _sget(starts_v, b) * CH
            nch = _sget(caps_v, b)

            @pl.when(nch > 0)
            def _():
                issue(st, 0, ebufA, idxA, dlocA, msgA, semA)

                def pair(ip, _):
                    c0 = ip * 2

                    @pl.when(c0 + 1 < nch)
                    def _():
                        issue(st, c0 + 1, ebufB, idxB, dlocB, msgB, semB)

                    drain(idxA, msgA, semA)
                    accum(dlocA, msgA)

                    @pl.when(c0 + 2 < nch)
                    def _():
                        issue(st, c0 + 2, ebufA, idxA, dlocA, msgA, semA)

                    @pl.when(c0 + 1 < nch)
                    def _():
                        drain(idxB, msgB, semB)
                        accum(dlocB, msgB)

                    return 0

                lax.fori_loop(0, (nch + 1) // 2, pair, 0)

            pltpu.sync_copy(dinv_hbm.at[pl.ds(b * BK, BK)], dinvbuf)
            for c in range(4):
                buf = msgA if c % 2 == 0 else msgB
                if c >= 2:
                    prows = pl.ds(b * BK + (c - 2) * CH, CH)
                    pltpu.make_async_copy(buf, p_hbm.at[prows], semW).wait()

                def rgrp(g, _, c=c, buf=buf):
                    dvvec = dinvbuf[pl.ds(c * CH + g * 16, 16)]
                    for l in range(16):
                        r = g * 16 + l
                        lr = c * CH + r
                        dv = jnp.full((16,), dvvec[l], jnp.float32)
                        sv = [slab[lr, pl.ds(j * 16, 16)] for j in range(8)]
                        for j in range(8):
                            buf[r, pl.ds(j * 16, 16)] = sv[j] * dv
                    return 0

                lax.fori_loop(0, CH // 16, rgrp, 0)
                rows = pl.ds(b * BK + c * CH, CH)
                pltpu.async_copy(buf, p_hbm.at[rows], semW)
            for c in range(2, 4):
                buf = msgA if c % 2 == 0 else msgB
                rows = pl.ds(b * BK + c * CH, CH)
                pltpu.make_async_copy(buf, p_hbm.at[rows], semW).wait()

        return 0

    lax.fori_loop(0, 4, bucket, 0)


def _sc_prop(packed_p, counts, g, dinv):
    return pl.kernel(
        _prop_body,
        out_type=jax.ShapeDtypeStruct((NPAD, 128), jnp.float32),
        mesh=plsc.VectorSubcoreMesh(**_MESH),
        **_CP,
        scratch_types=[
            pltpu.VMEM((NT, 128), jnp.int32),
            pltpu.VMEM((CH,), jnp.int32),
            pltpu.VMEM((CH,), jnp.int32),
            pltpu.VMEM((CH,), jnp.int32),
            pltpu.VMEM((CH,), jnp.int32),
            pltpu.VMEM((CH,), jnp.int32),
            pltpu.VMEM((CH,), jnp.int32),
            pltpu.VMEM((CH, 128), jnp.float32),
            pltpu.VMEM((CH, 128), jnp.float32),
            pltpu.VMEM((BK + 1, 128), jnp.float32),
            pltpu.VMEM((BK,), jnp.float32),
            pltpu.VMEM((128,), jnp.int32),
            pltpu.VMEM((128,), jnp.int32),
            pltpu.SemaphoreType.DMA,
            pltpu.SemaphoreType.DMA,
            pltpu.SemaphoreType.DMA,
        ],
    )(packed_p, counts, g, dinv)


def _t1_body(deg_ref, x_ref, dinv_ref, g0_ref, q0_ref):
    dv = lax.rsqrt(deg_ref[...])
    g0 = x_ref[...] * dv
    dinv_ref[...] = dv
    g0_ref[...] = g0
    q0_ref[...] = g0 * dv


def _tc_stage1(deg2, x_pad):
    return pl.pallas_call(
        _t1_body,
        grid=(NB,),
        in_specs=[
            pl.BlockSpec((BK, 1), lambda i: (i, 0)),
            pl.BlockSpec((BK, 128), lambda i: (i, 0)),
        ],
        out_specs=[
            pl.BlockSpec((BK, 1), lambda i: (i, 0)),
            pl.BlockSpec((BK, 128), lambda i: (i, 0)),
            pl.BlockSpec((BK, 128), lambda i: (i, 0)),
        ],
        out_shape=[
            jax.ShapeDtypeStruct((NPAD, 1), jnp.float32),
            jax.ShapeDtypeStruct((NPAD, 128), jnp.float32),
            jax.ShapeDtypeStruct((NPAD, 128), jnp.float32),
        ],
    )(deg2, x_pad)


def _t2_body(ps_ref, q_ref, w_ref, b_ref, dinv_ref, g_ref, qo_ref):
    p = ps_ref[...] + q_ref[...]
    h = jnp.dot(p, w_ref[...], preferred_element_type=jnp.float32)
    h = jnp.maximum(h + b_ref[...], 0.0)
    dv = dinv_ref[...]
    g = h * dv
    g_ref[...] = g
    qo_ref[...] = g * dv


def _tc_layer(ps, q, w, bvec, dinv2):
    return pl.pallas_call(
        _t2_body,
        grid=(NB,),
        in_specs=[
            pl.BlockSpec((BK, 128), lambda i: (i, 0)),
            pl.BlockSpec((BK, 128), lambda i: (i, 0)),
            pl.BlockSpec((128, 128), lambda i: (0, 0)),
            pl.BlockSpec((1, 128), lambda i: (0, 0)),
            pl.BlockSpec((BK, 1), lambda i: (i, 0)),
        ],
        out_specs=[
            pl.BlockSpec((BK, 128), lambda i: (i, 0)),
            pl.BlockSpec((BK, 128), lambda i: (i, 0)),
        ],
        out_shape=[
            jax.ShapeDtypeStruct((NPAD, 128), jnp.float32),
            jax.ShapeDtypeStruct((NPAD, 128), jnp.float32),
        ],
    )(ps, q, w, bvec, dinv2)


def _t4_body(ps_ref, q_ref, w2_ref, b2_ref, wc_ref, bc_ref, out_ref):
    p = ps_ref[...] + q_ref[...]
    h = jnp.dot(p, w2_ref[...], preferred_element_type=jnp.float32)
    h = jnp.maximum(h + b2_ref[...], 0.0)
    out_ref[...] = (
        jnp.dot(h, wc_ref[...], preferred_element_type=jnp.float32)
        + bc_ref[...]
    )


def _tc_final(ps2, q2, w2, b2v, wcp, bcp):
    return pl.pallas_call(
        _t4_body,
        grid=(NB,),
        in_specs=[
            pl.BlockSpec((BK, 128), lambda i: (i, 0)),
            pl.BlockSpec((BK, 128), lambda i: (i, 0)),
            pl.BlockSpec((128, 128), lambda i: (0, 0)),
            pl.BlockSpec((1, 128), lambda i: (0, 0)),
            pl.BlockSpec((128, 8), lambda i: (0, 0)),
            pl.BlockSpec((1, 8), lambda i: (0, 0)),
        ],
        out_specs=pl.BlockSpec((BK, 8), lambda i: (i, 0)),
        out_shape=jax.ShapeDtypeStruct((NPAD, 8), jnp.float32),
    )(ps2, q2, w2, b2v, wcp, bcp)


def kernel(x, edge_index, W1, b1, W2, b2, Wc, bc):
    src_e = edge_index[0]
    dst_e = edge_index[1]
    x_pad = jnp.pad(x, ((0, NPAD - N), (0, 128 - x.shape[1])))
    W1p = jnp.pad(W1, ((0, 128 - W1.shape[0]), (0, 0)))
    Wcp = jnp.pad(Wc, ((0, 0), (0, 8 - Wc.shape[1])))
    b1r = b1.reshape(1, 128)
    b2r = b2.reshape(1, 128)
    bcp = jnp.pad(bc, (0, 8 - bc.shape[0])).reshape(1, 8)

    counts, packed_e = _sc_hist(src_e, dst_e)
    packed_p = _sc_permute(packed_e, counts)
    deg = _sc_deg(packed_p, counts)
    dinv2, g0, q0 = _tc_stage1(deg.reshape(NPAD, 1), x_pad)
    dinv = dinv2.reshape(NPAD)

    ps0 = _sc_prop(packed_p, counts, g0, dinv)
    g1, q1 = _tc_layer(ps0, q0, W1p, b1r, dinv2)
    ps1 = _sc_prop(packed_p, counts, g1, dinv)
    g2, q2 = _tc_layer(ps1, q1, W2, b2r, dinv2)
    ps2 = _sc_prop(packed_p, counts, g2, dinv)
    out = _tc_final(ps2, q2, W2, b2r, Wcp, bcp)
    return out[:N, :2]
